# baseline (speedup 1.0000x reference)
import functools

import jax
import jax.numpy as jnp
from jax import lax
from jax.experimental import pallas as pl
from jax.experimental.pallas import tpu as pltpu
from jax.experimental.pallas import tpu_sc as plsc

EMB = 128
HID = 64
NC = 2
NS = 16
NW = NC * NS
L = 16
KV = HID // L
CHUNK = 128


def _precompute_tables(node_emb, wa, wb, ba):
    n = node_emb.shape[0]
    blk = 1000

    def pack_words(t):
        tb16 = t.astype(jnp.bfloat16)
        ub = jax.lax.bitcast_convert_type(
            tb16[:, :HID // 2], jnp.uint16).astype(jnp.uint32)
        vb = jax.lax.bitcast_convert_type(
            tb16[:, HID // 2:], jnp.uint16).astype(jnp.uint32)
        return jax.lax.bitcast_convert_type(
            ub | (vb << jnp.uint32(16)), jnp.int32)

    def body(emb_ref, wa_ref, wb_ref, ba_ref, a_ref, b_ref):
        e = emb_ref[...]
        a_ref[...] = pack_words(
            jnp.dot(e, wa_ref[...], preferred_element_type=jnp.float32)
            + ba_ref[...])
        b_ref[...] = pack_words(
            jnp.dot(e, wb_ref[...], preferred_element_type=jnp.float32))

    return pl.pallas_call(
        body,
        grid=(n // blk,),
        in_specs=[
            pl.BlockSpec((blk, EMB), lambda i: (i, 0)),
            pl.BlockSpec((EMB, HID), lambda i: (0, 0)),
            pl.BlockSpec((EMB, HID), lambda i: (0, 0)),
            pl.BlockSpec((1, HID), lambda i: (0, 0)),
        ],
        out_specs=[
            pl.BlockSpec((blk, HID // 2), lambda i: (i, 0)),
            pl.BlockSpec((blk, HID // 2), lambda i: (i, 0)),
        ],
        out_shape=[
            jax.ShapeDtypeStruct((n, HID // 2), jnp.int32),
            jax.ShapeDtypeStruct((n, HID // 2), jnp.int32),
        ],
    )(node_emb, wa, wb, ba)


def _make_sc_kernel(e_pad):
    ew = e_pad // NW
    nch = ew // CHUNK
    assert nch % 4 == 0

    mesh = plsc.VectorSubcoreMesh(
        core_axis_name="c", subcore_axis_name="s",
        num_cores=NC, num_subcores=NS,
    )

    @functools.partial(
        pl.kernel,
        out_type=jax.ShapeDtypeStruct((e_pad,), jnp.float32),
        mesh=mesh,
        compiler_params=pltpu.CompilerParams(
            needs_layout_passes=False, use_tc_tiling_on_sc=False),
        scratch_types=[
            pltpu.VMEM((ew,), jnp.int32),
            pltpu.VMEM((ew,), jnp.int32),
            pltpu.VMEM((ew,), jnp.float32),
            pltpu.VMEM((ew,), jnp.float32),
            pltpu.VMEM((CHUNK, HID // 2), jnp.int32),
            pltpu.VMEM((CHUNK, HID // 2), jnp.int32),
            pltpu.VMEM((CHUNK, HID // 2), jnp.int32),
            pltpu.VMEM((CHUNK, HID // 2), jnp.int32),
            pltpu.VMEM((CHUNK, HID // 2), jnp.int32),
            pltpu.VMEM((CHUNK, HID // 2), jnp.int32),
            pltpu.VMEM((CHUNK, HID // 2), jnp.int32),
            pltpu.VMEM((CHUNK, HID // 2), jnp.int32),
            pltpu.VMEM((KV, L), jnp.float32),
            pltpu.VMEM_SHARED((10000, HID // 2), jnp.int32),
            pltpu.VMEM_SHARED((10000, HID // 2), jnp.int32),
            pltpu.SemaphoreType.DMA,
            pltpu.SemaphoreType.DMA,
            pltpu.SemaphoreType.DMA,
            pltpu.SemaphoreType.DMA,
        ],
    )
    def sc_kernel(ta_hbm, tb_hbm, src_hbm, dst_hbm, noise_hbm, w2_hbm,
                  out_hbm, idx_s, idx_d, noise_v, out_v,
                  a0, a1, a2, a3, bb0, bb1, bb2, bb3, w2v, sh_a, sh_b,
                  sem0, sem1, sem2, sem3):
        wid = lax.axis_index("s") * NC + lax.axis_index("c")
        base = wid * ew
        sid = lax.axis_index("s")
        rows = 10000 // NS
        pltpu.sync_copy(ta_hbm.at[pl.ds(sid * rows, rows)],
                        sh_a.at[pl.ds(sid * rows, rows)])
        pltpu.sync_copy(tb_hbm.at[pl.ds(sid * rows, rows)],
                        sh_b.at[pl.ds(sid * rows, rows)])
        pltpu.sync_copy(src_hbm.at[pl.ds(base, ew)], idx_s)
        pltpu.sync_copy(dst_hbm.at[pl.ds(base, ew)], idx_d)
        pltpu.sync_copy(noise_hbm.at[pl.ds(base, ew)], noise_v)
        pltpu.sync_copy(w2_hbm, w2v)
        plsc.subcore_barrier()

        abufs = (a0, a1, a2, a3)
        bbufs = (bb0, bb1, bb2, bb3)
        sems = (sem0, sem1, sem2, sem3)
        nbuf = len(sems)
        last_lane = lax.iota(jnp.int32, L) == (L - 1)

        def fire(c, p):
            pltpu.async_copy(
                sh_a.at[idx_s.at[pl.ds(c * CHUNK, CHUNK)]], abufs[p], sems[p])
            pltpu.async_copy(
                sh_b.at[idx_d.at[pl.ds(c * CHUNK, CHUNK)]], bbufs[p], sems[p])

        def wait(c, p):
            pltpu.make_async_copy(
                sh_a.at[idx_s.at[pl.ds(c * CHUNK, CHUNK)]], abufs[p], sems[p]
            ).wait()
            pltpu.make_async_copy(
                sh_b.at[idx_d.at[pl.ds(c * CHUNK, CHUNK)]], bbufs[p], sems[p]
            ).wait()

        def compute(c, abuf, bbuf):
            w2 = [w2v[k, :] for k in range(KV)]

            def group(g, carry):
                eb = g * L
                opos = jnp.full((L,), c * CHUNK + eb, jnp.int32)
                for e in range(L):
                    row = eb + e
                    t = jnp.zeros((L,), jnp.float32)
                    for gg in range(KV // 2):
                        ap = plsc.bitcast(
                            abuf[row, pl.ds(gg * L, L)], jnp.bfloat16)
                        bp = plsc.bitcast(
                            bbuf[row, pl.ds(gg * L, L)], jnp.bfloat16)
                        a0, a1 = plsc.unpack(ap, format=plsc.PackFormat.INTERLEAVED)
                        b0, b1 = plsc.unpack(bp, format=plsc.PackFormat.INTERLEAVED)
                        t = t + jnp.maximum(a0 + b0, 0.0) * w2[gg * 2]
                        t = t + jnp.maximum(a1 + b1, 0.0) * w2[gg * 2 + 1]
                    cum = plsc.cumsum(t)
                    plsc.store_scatter(out_v, [opos + e], cum, mask=last_lane)
                return carry

            lax.fori_loop(0, CHUNK // L, group, 0)

        for p in range(nbuf):
            fire(p, p)

        def loop_body(kk, carry):
            for p in range(nbuf):
                c = kk * nbuf + p
                wait(c, p)
                compute(c, abufs[p], bbufs[p])

                @pl.when(c + nbuf < nch)
                def _():
                    fire(c + nbuf, p)

            return carry

        lax.fori_loop(0, nch // nbuf, loop_body, 0)

        def sig(i, carry):
            off = pl.ds(i * L, L)
            g = out_v[off] + noise_v[off]
            out_v[off] = 1.0 / (1.0 + jnp.exp(-g))
            return carry

        lax.fori_loop(0, ew // L, sig, 0)
        pltpu.sync_copy(out_v, out_hbm.at[pl.ds(base, ew)])

    return sc_kernel


def _threefry2x32_np(k1, k2, x0, x1):
    import numpy as np

    def rotl(x, d):
        return ((x << np.uint32(d)) | (x >> np.uint32(32 - d))).astype(np.uint32)

    rotations = [[13, 15, 26, 6], [17, 29, 16, 24]]
    ks = [np.uint32(k1), np.uint32(k2),
          np.uint32(k1) ^ np.uint32(k2) ^ np.uint32(0x1BD11BDA)]
    x0 = (x0 + ks[0]).astype(np.uint32)
    x1 = (x1 + ks[1]).astype(np.uint32)
    for r in range(5):
        for d in rotations[r % 2]:
            x0 = (x0 + x1).astype(np.uint32)
            x1 = x0 ^ rotl(x1, d)
        x0 = (x0 + ks[(r + 1) % 3]).astype(np.uint32)
        x1 = (x1 + ks[(r + 2) % 3] + np.uint32(r + 1)).astype(np.uint32)
    return x0, x1


@functools.cache
def _logistic_noise(half):
    import numpy as np
    bias = 0.0 + 0.0001
    idx = np.arange(half, dtype=np.uint64)
    c1 = (idx >> np.uint64(32)).astype(np.uint32)
    c2 = (idx & np.uint64(0xFFFFFFFF)).astype(np.uint32)
    b1, b2 = _threefry2x32_np(np.uint32(0), np.uint32(42), c1, c2)
    bits = b1 ^ b2
    flo = ((bits >> np.uint32(9)) | np.uint32(0x3F800000)).view(np.float32)
    m = flo - np.float32(1.0)
    span = np.float32(1.0 - bias) - np.float32(bias)
    eps = np.maximum(np.float32(bias), m * span + np.float32(bias))
    return np.log(eps) - np.log(np.float32(1.0) - eps)


def kernel(node_emb, edge_index, W1, b1, W2, b2):
    E = edge_index.shape[1]
    half = E // 2 - 1
    src = edge_index[0, :half]
    dst = edge_index[1, :half]

    noise = jnp.asarray(_logistic_noise(half)) + b2[0]

    import numpy as np
    inv = np.r_[0:16, 32:48, 16:32, 48:64]
    ta_i32, tb_i32 = _precompute_tables(
        node_emb, W1[:EMB][:, inv], W1[EMB:][:, inv], b1[inv].reshape(1, HID))

    grain = NW * CHUNK * 4
    e_pad = ((half + grain - 1) // grain) * grain
    pad = e_pad - half
    src_p = jnp.pad(src, (0, pad))
    dst_p = jnp.pad(dst, (0, pad))
    noise_p = jnp.pad(noise, (0, pad))
    w2rows = W2.reshape(KV, L)

    aug_pad = _make_sc_kernel(e_pad)(ta_i32, tb_i32, src_p, dst_p, noise_p,
                                     w2rows)
    aug = aug_pad[:half]

    sym_indices = jnp.concatenate(
        [edge_index[:, :half], edge_index[::-1, :half]], axis=1)
    sym_values = jnp.concatenate([aug, aug])
    return sym_indices, sym_values, aug


# speedup vs baseline: 1.0153x; 1.0153x over previous
import functools

import jax
import jax.numpy as jnp
from jax import lax
from jax.experimental import pallas as pl
from jax.experimental.pallas import tpu as pltpu
from jax.experimental.pallas import tpu_sc as plsc

EMB = 128
HID = 64
NC = 2
NS = 16
NW = NC * NS
L = 16
KV = HID // L
CHUNK = 128


def _precompute_tables(node_emb, wa, wb, ba):
    n = node_emb.shape[0]
    blk = 1000

    def pack_words(t):
        tb16 = t.astype(jnp.bfloat16)
        ub = jax.lax.bitcast_convert_type(
            tb16[:, :HID // 2], jnp.uint16).astype(jnp.uint32)
        vb = jax.lax.bitcast_convert_type(
            tb16[:, HID // 2:], jnp.uint16).astype(jnp.uint32)
        return jax.lax.bitcast_convert_type(
            ub | (vb << jnp.uint32(16)), jnp.int32)

    def body(emb_ref, wa_ref, wb_ref, ba_ref, a_ref, b_ref):
        e = emb_ref[...]
        a_ref[...] = pack_words(
            jnp.dot(e, wa_ref[...], preferred_element_type=jnp.float32)
            + ba_ref[...])
        b_ref[...] = pack_words(
            jnp.dot(e, wb_ref[...], preferred_element_type=jnp.float32))

    return pl.pallas_call(
        body,
        grid=(n // blk,),
        in_specs=[
            pl.BlockSpec((blk, EMB), lambda i: (i, 0)),
            pl.BlockSpec((EMB, HID), lambda i: (0, 0)),
            pl.BlockSpec((EMB, HID), lambda i: (0, 0)),
            pl.BlockSpec((1, HID), lambda i: (0, 0)),
        ],
        out_specs=[
            pl.BlockSpec((blk, HID // 2), lambda i: (i, 0)),
            pl.BlockSpec((blk, HID // 2), lambda i: (i, 0)),
        ],
        out_shape=[
            jax.ShapeDtypeStruct((n, HID // 2), jnp.int32),
            jax.ShapeDtypeStruct((n, HID // 2), jnp.int32),
        ],
    )(node_emb, wa, wb, ba)


def _make_sc_kernel(e_pad):
    ew = e_pad // NW
    nch = ew // CHUNK
    assert nch % 4 == 0

    mesh = plsc.VectorSubcoreMesh(
        core_axis_name="c", subcore_axis_name="s",
        num_cores=NC, num_subcores=NS,
    )

    @functools.partial(
        pl.kernel,
        out_type=jax.ShapeDtypeStruct((e_pad,), jnp.float32),
        mesh=mesh,
        compiler_params=pltpu.CompilerParams(
            needs_layout_passes=False, use_tc_tiling_on_sc=False),
        scratch_types=[
            pltpu.VMEM((ew,), jnp.int32),
            pltpu.VMEM((ew,), jnp.int32),
            pltpu.VMEM((ew,), jnp.float32),
            pltpu.VMEM((ew,), jnp.float32),
            pltpu.VMEM((CHUNK, HID // 2), jnp.int32),
            pltpu.VMEM((CHUNK, HID // 2), jnp.int32),
            pltpu.VMEM((CHUNK, HID // 2), jnp.int32),
            pltpu.VMEM((CHUNK, HID // 2), jnp.int32),
            pltpu.VMEM((CHUNK, HID // 2), jnp.int32),
            pltpu.VMEM((CHUNK, HID // 2), jnp.int32),
            pltpu.VMEM((CHUNK, HID // 2), jnp.int32),
            pltpu.VMEM((CHUNK, HID // 2), jnp.int32),
            pltpu.VMEM((KV, L), jnp.float32),
            pltpu.SemaphoreType.DMA,
            pltpu.SemaphoreType.DMA,
            pltpu.SemaphoreType.DMA,
            pltpu.SemaphoreType.DMA,
        ],
    )
    def sc_kernel(ta_hbm, tb_hbm, src_hbm, dst_hbm, noise_hbm, w2_hbm,
                  out_hbm, idx_s, idx_d, noise_v, out_v,
                  a0, a1, a2, a3, bb0, bb1, bb2, bb3, w2v,
                  sem0, sem1, sem2, sem3):
        wid = lax.axis_index("s") * NC + lax.axis_index("c")
        base = wid * ew
        pltpu.sync_copy(src_hbm.at[pl.ds(base, ew)], idx_s)
        pltpu.sync_copy(dst_hbm.at[pl.ds(base, ew)], idx_d)
        pltpu.sync_copy(noise_hbm.at[pl.ds(base, ew)], noise_v)
        pltpu.sync_copy(w2_hbm, w2v)

        abufs = (a0, a1, a2, a3)
        bbufs = (bb0, bb1, bb2, bb3)
        sems = (sem0, sem1, sem2, sem3)
        nbuf = len(sems)
        last_lane = lax.iota(jnp.int32, L) == (L - 1)

        def fire(c, p):
            pltpu.async_copy(
                ta_hbm.at[idx_s.at[pl.ds(c * CHUNK, CHUNK)]], abufs[p], sems[p])
            pltpu.async_copy(
                tb_hbm.at[idx_d.at[pl.ds(c * CHUNK, CHUNK)]], bbufs[p], sems[p])

        def wait(c, p):
            pltpu.make_async_copy(
                ta_hbm.at[idx_s.at[pl.ds(c * CHUNK, CHUNK)]], abufs[p], sems[p]
            ).wait()
            pltpu.make_async_copy(
                tb_hbm.at[idx_d.at[pl.ds(c * CHUNK, CHUNK)]], bbufs[p], sems[p]
            ).wait()

        def compute(c, abuf, bbuf):
            w2 = [w2v[k, :] for k in range(KV)]

            def group(g, carry):
                eb = g * L
                opos = jnp.full((L,), c * CHUNK + eb, jnp.int32)
                for e in range(L):
                    row = eb + e
                    t = jnp.zeros((L,), jnp.float32)
                    for gg in range(KV // 2):
                        ap = plsc.bitcast(
                            abuf[row, pl.ds(gg * L, L)], jnp.bfloat16)
                        bp = plsc.bitcast(
                            bbuf[row, pl.ds(gg * L, L)], jnp.bfloat16)
                        a0, a1 = plsc.unpack(ap, format=plsc.PackFormat.INTERLEAVED)
                        b0, b1 = plsc.unpack(bp, format=plsc.PackFormat.INTERLEAVED)
                        t = t + jnp.maximum(a0 + b0, 0.0) * w2[gg * 2]
                        t = t + jnp.maximum(a1 + b1, 0.0) * w2[gg * 2 + 1]
                    cum = plsc.cumsum(t)
                    plsc.store_scatter(out_v, [opos + e], cum, mask=last_lane)
                return carry

            lax.fori_loop(0, CHUNK // L, group, 0)

        for p in range(nbuf):
            fire(p, p)

        def loop_body(kk, carry):
            for p in range(nbuf):
                c = kk * nbuf + p
                wait(c, p)
                compute(c, abufs[p], bbufs[p])

                @pl.when(c + nbuf < nch)
                def _():
                    fire(c + nbuf, p)

            return carry

        lax.fori_loop(0, nch // nbuf, loop_body, 0)

        def sig(i, carry):
            off = pl.ds(i * L, L)
            g = out_v[off] + noise_v[off]
            out_v[off] = 1.0 / (1.0 + jnp.exp(-g))
            return carry

        lax.fori_loop(0, ew // L, sig, 0)
        pltpu.sync_copy(out_v, out_hbm.at[pl.ds(base, ew)])

    return sc_kernel


def _threefry2x32_np(k1, k2, x0, x1):
    import numpy as np

    def rotl(x, d):
        return ((x << np.uint32(d)) | (x >> np.uint32(32 - d))).astype(np.uint32)

    rotations = [[13, 15, 26, 6], [17, 29, 16, 24]]
    ks = [np.uint32(k1), np.uint32(k2),
          np.uint32(k1) ^ np.uint32(k2) ^ np.uint32(0x1BD11BDA)]
    x0 = (x0 + ks[0]).astype(np.uint32)
    x1 = (x1 + ks[1]).astype(np.uint32)
    for r in range(5):
        for d in rotations[r % 2]:
            x0 = (x0 + x1).astype(np.uint32)
            x1 = x0 ^ rotl(x1, d)
        x0 = (x0 + ks[(r + 1) % 3]).astype(np.uint32)
        x1 = (x1 + ks[(r + 2) % 3] + np.uint32(r + 1)).astype(np.uint32)
    return x0, x1


@functools.cache
def _logistic_noise(half):
    import numpy as np
    bias = 0.0 + 0.0001
    idx = np.arange(half, dtype=np.uint64)
    c1 = (idx >> np.uint64(32)).astype(np.uint32)
    c2 = (idx & np.uint64(0xFFFFFFFF)).astype(np.uint32)
    b1, b2 = _threefry2x32_np(np.uint32(0), np.uint32(42), c1, c2)
    bits = b1 ^ b2
    flo = ((bits >> np.uint32(9)) | np.uint32(0x3F800000)).view(np.float32)
    m = flo - np.float32(1.0)
    span = np.float32(1.0 - bias) - np.float32(bias)
    eps = np.maximum(np.float32(bias), m * span + np.float32(bias))
    return np.log(eps) - np.log(np.float32(1.0) - eps)


def kernel(node_emb, edge_index, W1, b1, W2, b2):
    E = edge_index.shape[1]
    half = E // 2 - 1
    src = edge_index[0, :half]
    dst = edge_index[1, :half]

    noise = jnp.asarray(_logistic_noise(half)) + b2[0]

    import numpy as np
    inv = np.r_[0:16, 32:48, 16:32, 48:64]
    ta_i32, tb_i32 = _precompute_tables(
        node_emb, W1[:EMB][:, inv], W1[EMB:][:, inv], b1[inv].reshape(1, HID))

    grain = NW * CHUNK * 4
    e_pad = ((half + grain - 1) // grain) * grain
    pad = e_pad - half
    src_p = jnp.pad(src, (0, pad))
    dst_p = jnp.pad(dst, (0, pad))
    noise_p = jnp.pad(noise, (0, pad))
    w2rows = W2.reshape(KV, L)

    aug_pad = _make_sc_kernel(e_pad)(ta_i32, tb_i32, src_p, dst_p, noise_p,
                                     w2rows)
    aug = aug_pad[:half]

    sym_indices = jnp.concatenate(
        [edge_index[:, :half], edge_index[::-1, :half]], axis=1)
    sym_values = jnp.concatenate([aug, aug])
    return sym_indices, sym_values, aug
